# baseline (speedup 1.0000x reference)
import functools

import jax
import jax.numpy as jnp
from jax import lax
from jax.experimental import pallas as pl
from jax.experimental.pallas import tpu as pltpu
from jax.experimental.pallas import tpu_sc as plsc

_N_E = 8192
_E = 32
_SRC = 1024
_NT = 8192
_TB = 1024
_KC = 256
_BETA = 0.25

_NW = 32
_BPW = _NT // _NW


def _enc_kernel(x_ref, we_ref, be_ref, z_ref):
    acc = jnp.zeros((_TB, _E), jnp.float32)
    for k0 in range(0, _SRC, _KC):
        acc = acc + jnp.dot(x_ref[:, k0:k0 + _KC], we_ref[k0:k0 + _KC, :],
                            preferred_element_type=jnp.float32)
    z_ref[...] = acc + be_ref[...]


def _encode(xf, W_enc, b_enc):
    return pl.pallas_call(
        _enc_kernel,
        grid=(_NT // _TB,),
        in_specs=[
            pl.BlockSpec((_TB, _SRC), lambda i: (i, 0)),
            pl.BlockSpec((_SRC, _E), lambda i: (0, 0)),
            pl.BlockSpec((1, _E), lambda i: (0, 0)),
        ],
        out_specs=pl.BlockSpec((_TB, _E), lambda i: (i, 0)),
        out_shape=jax.ShapeDtypeStruct((_NT, _E), jnp.float32),
    )(xf, W_enc, b_enc.reshape(1, _E))


_sc_mesh = plsc.VectorSubcoreMesh(core_axis_name="c", subcore_axis_name="s")


@functools.partial(
    pl.kernel,
    mesh=_sc_mesh,
    out_type=jax.ShapeDtypeStruct((_NT, 128), jnp.float32),
    scratch_types=[
        pltpu.VMEM((2, 128), jnp.int32),
        pltpu.VMEM((_BPW, 128), jnp.float32),
        pltpu.SemaphoreType.DMA,
    ],
)
def _sc_gather(table_hbm, idx_hbm, out_hbm, idx_v, rows_v, sem):
    wid = lax.axis_index("s") * 2 + lax.axis_index("c")
    base = wid * _BPW
    for j in range(2):
        pltpu.sync_copy(idx_hbm.at[pl.ds(base + j * 128, 128)], idx_v.at[j])
        pltpu.async_copy(table_hbm.at[idx_v.at[j]],
                         rows_v.at[pl.ds(j * 128, 128), :], sem).wait()
    pltpu.sync_copy(rows_v, out_hbm.at[pl.ds(base, _BPW)])


def _dec_kernel(zq_ref, z_ref, wd_ref, bd_ref, out_ref, loss_ref):
    i = pl.program_id(0)
    z = z_ref[...]
    zq = zq_ref[:, :_E]
    dz = zq - z
    ste = z + dz
    out_ref[...] = (jnp.dot(ste, wd_ref[...], preferred_element_type=jnp.float32)
                    + bd_ref[...])
    part = jnp.sum(dz * dz, keepdims=True).reshape(1, 1)
    loss_ref[...] = jnp.where(i == 0, part, loss_ref[...] + part)


def _decode(zq, z_flat, W_dec, b_dec):
    return pl.pallas_call(
        _dec_kernel,
        grid=(_NT // _TB,),
        in_specs=[
            pl.BlockSpec((_TB, 128), lambda i: (i, 0)),
            pl.BlockSpec((_TB, _E), lambda i: (i, 0)),
            pl.BlockSpec((_E, _SRC), lambda i: (0, 0)),
            pl.BlockSpec((1, _SRC), lambda i: (0, 0)),
        ],
        out_specs=[
            pl.BlockSpec((_TB, _SRC), lambda i: (i, 0)),
            pl.BlockSpec((1, 1), lambda i: (0, 0)),
        ],
        out_shape=[
            jax.ShapeDtypeStruct((_NT, _SRC), jnp.float32),
            jax.ShapeDtypeStruct((1, 1), jnp.float32),
        ],
    )(zq, z_flat, W_dec, b_dec.reshape(1, _SRC))


def kernel(x, W_enc, b_enc, codebook, W_dec, b_dec):
    B, T, D = x.shape
    z_flat = _encode(x.reshape(_NT, _SRC), W_enc, b_enc)
    dm = (jnp.sum(z_flat ** 2, axis=1, keepdims=True)
          + jnp.sum(codebook ** 2, axis=1)
          - 2.0 * jnp.einsum('bd,dn->bn', z_flat, jnp.transpose(codebook)))
    idx = jnp.argmin(dm, axis=1).reshape(B, T)
    z_q = jnp.take(codebook, idx, axis=0).reshape(_NT, _E)
    zq_pad = jnp.pad(z_q, ((0, 0), (0, 128 - _E)))
    out_f, loss_sum = _decode(zq_pad, z_flat, W_dec, b_dec)
    m = loss_sum[0, 0] / (_NT * _E)
    loss = _BETA * m + m
    return out_f.reshape(B, T, D), loss


# speedup vs baseline: 1.0057x; 1.0057x over previous
import functools

import jax
import jax.numpy as jnp
from jax import lax
from jax.experimental import pallas as pl
from jax.experimental.pallas import tpu as pltpu
from jax.experimental.pallas import tpu_sc as plsc

_N_E = 8192
_E = 32
_SRC = 1024
_NT = 8192
_TB = 1024
_KC = 256
_BETA = 0.25

_NW = 32
_BPW = _NT // _NW


def _enc_kernel(x_ref, we_ref, be_ref, z_ref):
    acc = jnp.zeros((_TB, _E), jnp.float32)
    for k0 in range(0, _SRC, _KC):
        acc = acc + jnp.dot(x_ref[:, k0:k0 + _KC], we_ref[k0:k0 + _KC, :],
                            preferred_element_type=jnp.float32)
    z_ref[...] = acc + be_ref[...]


def _encode(xf, W_enc, b_enc):
    return pl.pallas_call(
        _enc_kernel,
        grid=(_NT // _TB,),
        in_specs=[
            pl.BlockSpec((_TB, _SRC), lambda i: (i, 0)),
            pl.BlockSpec((_SRC, _E), lambda i: (0, 0)),
            pl.BlockSpec((1, _E), lambda i: (0, 0)),
        ],
        out_specs=pl.BlockSpec((_TB, _E), lambda i: (i, 0)),
        out_shape=jax.ShapeDtypeStruct((_NT, _E), jnp.float32),
    )(xf, W_enc, b_enc.reshape(1, _E))


_sc_mesh = plsc.VectorSubcoreMesh(core_axis_name="c", subcore_axis_name="s")


@functools.partial(
    pl.kernel,
    mesh=_sc_mesh,
    out_type=jax.ShapeDtypeStruct((_NT, 128), jnp.float32),
    scratch_types=[
        pltpu.VMEM((2, 128), jnp.int32),
        pltpu.VMEM((_BPW, 128), jnp.float32),
        pltpu.SemaphoreType.DMA,
    ],
)
def _sc_gather(table_hbm, idx_hbm, out_hbm, idx_v, rows_v, sem):
    wid = lax.axis_index("s") * 2 + lax.axis_index("c")
    base = wid * _BPW
    for j in range(2):
        pltpu.sync_copy(idx_hbm.at[pl.ds(base + j * 128, 128)], idx_v.at[j])
        pltpu.async_copy(table_hbm.at[idx_v.at[j]],
                         rows_v.at[pl.ds(j * 128, 128), :], sem).wait()
    pltpu.sync_copy(rows_v, out_hbm.at[pl.ds(base, _BPW)])


def _dec_kernel(zq_ref, z_ref, wd_ref, bd_ref, out_ref, loss_ref):
    i = pl.program_id(0)
    z = z_ref[...]
    zq = zq_ref[...]
    dz = zq - z
    ste = z + dz
    out_ref[...] = (jnp.dot(ste, wd_ref[...], preferred_element_type=jnp.float32)
                    + bd_ref[...])
    part = jnp.sum(dz * dz, keepdims=True).reshape(1, 1)
    loss_ref[...] = jnp.where(i == 0, part, loss_ref[...] + part)


def _decode(zq, z_flat, W_dec, b_dec):
    return pl.pallas_call(
        _dec_kernel,
        grid=(_NT // _TB,),
        in_specs=[
            pl.BlockSpec((_TB, _E), lambda i: (i, 0)),
            pl.BlockSpec((_TB, _E), lambda i: (i, 0)),
            pl.BlockSpec((_E, _SRC), lambda i: (0, 0)),
            pl.BlockSpec((1, _SRC), lambda i: (0, 0)),
        ],
        out_specs=[
            pl.BlockSpec((_TB, _SRC), lambda i: (i, 0)),
            pl.BlockSpec((1, 1), lambda i: (0, 0)),
        ],
        out_shape=[
            jax.ShapeDtypeStruct((_NT, _SRC), jnp.float32),
            jax.ShapeDtypeStruct((1, 1), jnp.float32),
        ],
    )(zq, z_flat, W_dec, b_dec.reshape(1, _SRC))


def kernel(x, W_enc, b_enc, codebook, W_dec, b_dec):
    B, T, D = x.shape
    z_flat = _encode(x.reshape(_NT, _SRC), W_enc, b_enc)
    dm = (jnp.sum(z_flat ** 2, axis=1, keepdims=True)
          + jnp.sum(codebook ** 2, axis=1)
          - 2.0 * jnp.einsum('bd,dn->bn', z_flat, jnp.transpose(codebook)))
    idx = jnp.argmin(dm, axis=1).reshape(B, T)
    z_q = jnp.take(codebook, idx, axis=0).reshape(_NT, _E)
    out_f, loss_sum = _decode(z_q, z_flat, W_dec, b_dec)
    m = loss_sum[0, 0] / (_NT * _E)
    loss = _BETA * m + m
    return out_f.reshape(B, T, D), loss
